# baseline (speedup 1.0000x reference)
import functools
import jax
import jax.numpy as jnp
from jax.experimental import pallas as pl
from jax.experimental.pallas import tpu as pltpu


_BLK = 5000


def _gram_body(x_ref, gram_ref, csum_ref):
    i = pl.program_id(0)
    x = x_ref[...]
    g = jax.lax.dot_general(x, x, (((0,), (0,)), ((), ())),
                            preferred_element_type=jnp.float32)
    s = jnp.sum(x, axis=0, keepdims=True)

    @pl.when(i == 0)
    def _():
        gram_ref[...] = jnp.zeros_like(gram_ref)
        csum_ref[...] = jnp.zeros_like(csum_ref)

    gram_ref[...] += g
    csum_ref[...] += s


def _gram_colsum(x, blk):
    n, c = x.shape
    nb = n // blk
    return pl.pallas_call(
        _gram_body,
        grid=(nb,),
        in_specs=[pl.BlockSpec((blk, c), lambda i: (i, 0))],
        out_specs=(pl.BlockSpec((c, c), lambda i: (0, 0)),
                   pl.BlockSpec((1, c), lambda i: (0, 0))),
        out_shape=(jax.ShapeDtypeStruct((c, c), jnp.float32),
                   jax.ShapeDtypeStruct((1, c), jnp.float32)),
    )(x)


def _lin_bn_stats(x, Wt, b):
    n = x.shape[0]
    gram, csum = _gram_colsum(x, _BLK if x.shape[0] % _BLK == 0 else x.shape[0])
    mean = (csum / n) @ Wt + b
    e2 = jnp.sum(Wt * (gram @ Wt), axis=0) / n
    e2 = e2 + 2.0 * b * ((csum / n) @ Wt) + b * b
    var = e2 - mean * mean
    return mean, var


def _bn_fwd_body(x_ref, w1t_ref, a1_ref, c1_ref, wqt_ref, bq_ref,
                 h1_ref, xq_ref):
    h1 = jnp.maximum(
        jnp.dot(x_ref[...], w1t_ref[...], preferred_element_type=jnp.float32)
        * a1_ref[...] + c1_ref[...], 0.0)
    h1_ref[...] = h1
    xq_ref[...] = jnp.dot(h1, wqt_ref[...],
                          preferred_element_type=jnp.float32) + bq_ref[...]


def _h1_xq(x1, W1t, a1, c1, Wqt, bq):
    n, cin = x1.shape
    c = W1t.shape[1]
    nb = n // _BLK
    return pl.pallas_call(
        _bn_fwd_body,
        grid=(nb,),
        in_specs=[
            pl.BlockSpec((_BLK, cin), lambda i: (i, 0)),
            pl.BlockSpec((cin, c), lambda i: (0, 0)),
            pl.BlockSpec((1, c), lambda i: (0, 0)),
            pl.BlockSpec((1, c), lambda i: (0, 0)),
            pl.BlockSpec((c, c), lambda i: (0, 0)),
            pl.BlockSpec((1, c), lambda i: (0, 0)),
        ],
        out_specs=(pl.BlockSpec((_BLK, c), lambda i: (i, 0)),
                   pl.BlockSpec((_BLK, c), lambda i: (i, 0))),
        out_shape=(jax.ShapeDtypeStruct((n, c), jnp.float32),
                   jax.ShapeDtypeStruct((n, c), jnp.float32)),
    )(x1, W1t, a1, c1, Wqt, bq)


def _h2_body(x2_ref, w2t_ref, b2_ref, g2_ref, be2_ref, h2_ref):
    z2 = jnp.dot(x2_ref[...], w2t_ref[...],
                 preferred_element_type=jnp.float32) + b2_ref[...]
    m2 = jnp.mean(z2, axis=0, keepdims=True)
    v2 = jnp.var(z2, axis=0, keepdims=True)
    h2_ref[...] = jnp.maximum(
        (z2 - m2) * (g2_ref[...] * jax.lax.rsqrt(v2 + 1e-5)) + be2_ref[...], 0.0)


def _dense_pre(x1, x2, W1, b1, g1, be1, W2, b2, g2, be2, Wq, bq):
    n1, c = x1.shape[0], W1.shape[0]
    n2 = x2.shape[0]
    r2 = lambda a: a.reshape(1, -1)
    m1, v1 = _lin_bn_stats(x1, W1.T, r2(b1))
    a1 = r2(g1) * jax.lax.rsqrt(v1 + 1e-5)
    c1 = r2(be1) - m1 * a1 + r2(b1) * a1
    h1, xq = _h1_xq(x1, W1.T, a1, c1, Wq.T, r2(bq))
    h2 = pl.pallas_call(
        _h2_body,
        out_shape=jax.ShapeDtypeStruct((n2, c), jnp.float32),
    )(x2, W2.T, r2(b2), r2(g2), r2(be2))
    return h1, xq, h2


def kernel(p1, x1, p2, x2, knn_idx, interp_idx, W1, b1, g1, be1, W2, b2, g2,
           be2, Wq, bq, Wk, bk, Wv, bv, gw0, bw0, Ww1, bw1, gw1, bew1, Ww2,
           bw2):
    h1, xq, h2 = _dense_pre(x1, x2, W1, b1, g1, be1, W2, b2, g2, be2, Wq, bq)

    nb = jnp.take(p2, interp_idx, axis=0)
    dist = jnp.sqrt(jnp.sum((p1[:, None, :] - nb) ** 2, axis=-1))
    iw = 1.0 / (dist + 1e-8)
    iw = iw / jnp.sum(iw, axis=-1, keepdims=True)
    x2i = jnp.sum(jnp.take(h2, interp_idx, axis=0) * iw[:, :, None], axis=1)

    xk = x2i @ Wk.T + bk
    xv = x2i @ Wv.T + bv

    xk_g = jnp.take(xk, knn_idx, axis=0)
    xv_g = jnp.take(xv, knn_idx, axis=0)
    w = xk_g - xq[:, None, :]
    m0 = jnp.mean(w, axis=(0, 1))
    v0 = jnp.var(w, axis=(0, 1))
    w = jnp.maximum((w - m0) * (gw0 * jax.lax.rsqrt(v0 + 1e-5)) + bw0, 0.0)
    w = w @ Ww1.T + bw1
    m1 = jnp.mean(w, axis=(0, 1))
    v1 = jnp.var(w, axis=(0, 1))
    w = jnp.maximum((w - m1) * (gw1 * jax.lax.rsqrt(v1 + 1e-5)) + bew1, 0.0)
    w = w @ Ww2.T + bw2
    w = jax.nn.softmax(w, axis=1)
    n, ns, c = xv_g.shape
    s = c // w.shape[-1]
    x2o = jnp.sum(xv_g.reshape(n, ns, s, c // s) * w[:, :, None, :],
                  axis=1).reshape(n, c)
    return h1 + x2o


# speedup vs baseline: 1.0065x; 1.0065x over previous
import functools
import jax
import jax.numpy as jnp
from jax import lax
from jax.experimental import pallas as pl
from jax.experimental.pallas import tpu as pltpu
from jax.experimental.pallas import tpu_sc as plsc

N1 = 50000
N2 = 12500
C = 64
NS = 16
S = 8

_BLK = 5000
_QB = 400
_CH = 128

_INFO = plsc.get_sparse_core_info()
_NW = _INFO.num_cores * _INFO.num_subcores


def _sc_gather(table, idx):
    b = idx.shape[0]
    d = table.shape[1]
    per_w = b // _NW
    nch = per_w // _CH
    mesh = plsc.VectorSubcoreMesh(core_axis_name="c", subcore_axis_name="s")

    @functools.partial(
        pl.kernel, mesh=mesh,
        out_type=jax.ShapeDtypeStruct((b, d), jnp.float32),
        compiler_params=pltpu.CompilerParams(use_tc_tiling_on_sc=False),
        scratch_types=[
            pltpu.VMEM((_CH,), jnp.int32),
            pltpu.VMEM((_CH, d), jnp.float32),
            pltpu.SemaphoreType.DMA,
        ],
    )
    def k(table_hbm, idx_hbm, out_hbm, idx_v, rows_v, sem):
        wid = lax.axis_index("s") * _INFO.num_cores + lax.axis_index("c")
        base = wid * per_w

        def body(t, carry):
            off = base + t * _CH
            pltpu.sync_copy(idx_hbm.at[pl.ds(off, _CH)], idx_v)
            pltpu.async_copy(table_hbm.at[idx_v], rows_v, sem).wait()
            pltpu.sync_copy(rows_v, out_hbm.at[pl.ds(off, _CH)])
            return carry

        lax.fori_loop(0, nch, body, 0)

    return k(table, idx)


def _pad_idx(idx_flat):
    b = idx_flat.shape[0]
    unit = _NW * _CH
    b_pad = ((b + unit - 1) // unit) * unit
    return jnp.pad(idx_flat, (0, b_pad - b))


def _gram_body(x_ref, gram_ref, csum_ref):
    i = pl.program_id(0)
    x = x_ref[...]
    g = lax.dot_general(x, x, (((0,), (0,)), ((), ())),
                        preferred_element_type=jnp.float32)
    s = jnp.sum(x, axis=0, keepdims=True)

    @pl.when(i == 0)
    def _():
        gram_ref[...] = jnp.zeros_like(gram_ref)
        csum_ref[...] = jnp.zeros_like(csum_ref)

    gram_ref[...] += g
    csum_ref[...] += s


def _lin_bn_stats(x, Wt, b):
    n, c = x.shape
    nb = n // _BLK
    gram, csum = pl.pallas_call(
        _gram_body,
        grid=(nb,),
        in_specs=[pl.BlockSpec((_BLK, c), lambda i: (i, 0))],
        out_specs=(pl.BlockSpec((c, c), lambda i: (0, 0)),
                   pl.BlockSpec((1, c), lambda i: (0, 0))),
        out_shape=(jax.ShapeDtypeStruct((c, c), jnp.float32),
                   jax.ShapeDtypeStruct((1, c), jnp.float32)),
    )(x)
    mean = (csum / n) @ Wt + b
    e2 = jnp.sum(Wt * (gram @ Wt), axis=0)[None] / n
    e2 = e2 + 2.0 * b * ((csum / n) @ Wt) + b * b
    var = e2 - mean * mean
    return mean, var


def _bn_fwd_body(x_ref, w1t_ref, a1_ref, c1_ref, wqt_ref, bq_ref,
                 h1_ref, xq_ref):
    h1 = jnp.maximum(
        jnp.dot(x_ref[...], w1t_ref[...], preferred_element_type=jnp.float32)
        * a1_ref[...] + c1_ref[...], 0.0)
    h1_ref[...] = h1
    xq_ref[...] = jnp.dot(h1, wqt_ref[...],
                          preferred_element_type=jnp.float32) + bq_ref[...]


def _h2_body(x2_ref, w2t_ref, b2_ref, g2_ref, be2_ref, h2_ref):
    z2 = jnp.dot(x2_ref[...], w2t_ref[...],
                 preferred_element_type=jnp.float32) + b2_ref[...]
    m2 = jnp.mean(z2, axis=0, keepdims=True)
    v2 = jnp.mean(z2 * z2, axis=0, keepdims=True) - m2 * m2
    h2_ref[...] = jnp.maximum(
        (z2 - m2) * (g2_ref[...] * lax.rsqrt(v2 + 1e-5)) + be2_ref[...], 0.0)


def _dense_pre(x1, x2, W1, b1, g1, be1, W2, b2, g2, be2, Wq, bq):
    n1, cin = x1.shape
    n2 = x2.shape[0]
    r2 = lambda a: a.reshape(1, -1)
    m1, v1 = _lin_bn_stats(x1, W1.T, r2(b1))
    a1 = r2(g1) * lax.rsqrt(v1 + 1e-5)
    c1 = r2(be1) - m1 * a1 + r2(b1) * a1
    nb = n1 // _BLK
    h1, xq = pl.pallas_call(
        _bn_fwd_body,
        grid=(nb,),
        in_specs=[
            pl.BlockSpec((_BLK, cin), lambda i: (i, 0)),
            pl.BlockSpec((cin, C), lambda i: (0, 0)),
            pl.BlockSpec((1, C), lambda i: (0, 0)),
            pl.BlockSpec((1, C), lambda i: (0, 0)),
            pl.BlockSpec((C, C), lambda i: (0, 0)),
            pl.BlockSpec((1, C), lambda i: (0, 0)),
        ],
        out_specs=(pl.BlockSpec((_BLK, C), lambda i: (i, 0)),
                   pl.BlockSpec((_BLK, C), lambda i: (i, 0))),
        out_shape=(jax.ShapeDtypeStruct((n1, C), jnp.float32),
                   jax.ShapeDtypeStruct((n1, C), jnp.float32)),
    )(x1, W1.T, a1, c1, Wq.T, r2(bq))
    h2 = pl.pallas_call(
        _h2_body,
        out_shape=jax.ShapeDtypeStruct((n2, C), jnp.float32),
    )(x2, W2.T, r2(b2), r2(g2), r2(be2))
    return h1, xq, h2


def _interp_body(g_ref, p1_ref, wkt_ref, bk_ref, wvt_ref, bv_ref,
                 xk_ref, xv_ref):
    g = g_ref[...].reshape(_QB, 3, 80)
    h2n = g[:, :, :C]
    p2n = g[:, :, C:C + 3]
    d = jnp.sqrt(jnp.sum((p1_ref[...][:, None, :] - p2n) ** 2, axis=-1))
    iw = 1.0 / (d + 1e-8)
    iw = iw / jnp.sum(iw, axis=-1, keepdims=True)
    x2i = jnp.sum(h2n * iw[:, :, None], axis=1)
    xk_ref[...] = jnp.dot(x2i, wkt_ref[...],
                          preferred_element_type=jnp.float32) + bk_ref[...]
    xv_ref[...] = jnp.dot(x2i, wvt_ref[...],
                          preferred_element_type=jnp.float32) + bv_ref[...]


def _interp_kv(g, p1, Wk, bk, Wv, bv):
    nb = N1 // _QB
    r2 = lambda a: a.reshape(1, -1)
    return pl.pallas_call(
        _interp_body,
        grid=(nb,),
        in_specs=[
            pl.BlockSpec((3 * _QB, 80), lambda i: (i, 0)),
            pl.BlockSpec((_QB, 3), lambda i: (i, 0)),
            pl.BlockSpec((C, C), lambda i: (0, 0)),
            pl.BlockSpec((1, C), lambda i: (0, 0)),
            pl.BlockSpec((C, C), lambda i: (0, 0)),
            pl.BlockSpec((1, C), lambda i: (0, 0)),
        ],
        out_specs=(pl.BlockSpec((_QB, C), lambda i: (i, 0)),
                   pl.BlockSpec((_QB, C), lambda i: (i, 0))),
        out_shape=(jax.ShapeDtypeStruct((N1, C), jnp.float32),
                   jax.ShapeDtypeStruct((N1, C), jnp.float32)),
    )(g, p1, Wk.T, r2(bk), Wv.T, r2(bv))


def _wstats_body(kv_ref, xq_ref, s_ref, ss_ref):
    i = pl.program_id(0)
    w = kv_ref[...].reshape(_QB, NS, C) - xq_ref[...][:, None, :]

    @pl.when(i == 0)
    def _():
        s_ref[...] = jnp.zeros_like(s_ref)
        ss_ref[...] = jnp.zeros_like(ss_ref)

    s_ref[...] += jnp.sum(w, axis=(0, 1))[None]
    ss_ref[...] += jnp.sum(w * w, axis=(0, 1))[None]


def _y_body(kv_ref, xq_ref, a0_ref, c0_ref, w1t_ref, bw1_ref,
            y_ref, s_ref, ss_ref):
    i = pl.program_id(0)
    w = kv_ref[...].reshape(_QB, NS, C) - xq_ref[...][:, None, :]
    u = jnp.maximum(w * a0_ref[...] + c0_ref[...], 0.0)
    y = jnp.dot(u.reshape(_QB * NS, C), w1t_ref[...],
                preferred_element_type=jnp.float32) + bw1_ref[...]
    y_ref[...] = y

    @pl.when(i == 0)
    def _():
        s_ref[...] = jnp.zeros_like(s_ref)
        ss_ref[...] = jnp.zeros_like(ss_ref)

    s_ref[...] += jnp.sum(y, axis=0, keepdims=True)
    ss_ref[...] += jnp.sum(y * y, axis=0, keepdims=True)


def _out_body(y_ref, kv_ref, h1_ref, a1_ref, c1_ref, w2t_ref, bw2_ref, o_ref):
    v = jnp.maximum(y_ref[...] * a1_ref[...] + c1_ref[...], 0.0)
    z = jnp.dot(v, w2t_ref[...],
                preferred_element_type=jnp.float32) + bw2_ref[...]
    z = z.reshape(_QB, NS, C // S)
    z = z - jnp.max(z, axis=1, keepdims=True)
    e = jnp.exp(z)
    wsm = e / jnp.sum(e, axis=1, keepdims=True)
    xv = kv_ref[...].reshape(_QB, NS, S, C // S)
    x2o = jnp.sum(xv * wsm[:, :, None, :], axis=1).reshape(_QB, C)
    o_ref[...] = h1_ref[...] + x2o


def _attention(xk_g, xv_g, xq, h1, gw0, bw0, Ww1, bw1, gw1, bew1, Ww2, bw2):
    nb = N1 // _QB
    r2 = lambda a: a.reshape(1, -1)
    cnt = float(N1 * NS)

    kv_spec_k = pl.BlockSpec((NS * _QB, C), lambda i: (i, 0))
    kv_spec_v = pl.BlockSpec((NS * _QB, C), lambda i: (i, 0))
    xq_spec = pl.BlockSpec((_QB, C), lambda i: (i, 0))
    c_spec = lambda c: pl.BlockSpec((1, c), lambda i: (0, 0))

    s0, ss0 = pl.pallas_call(
        _wstats_body,
        grid=(nb,),
        in_specs=[kv_spec_k, xq_spec],
        out_specs=(c_spec(C), c_spec(C)),
        out_shape=(jax.ShapeDtypeStruct((1, C), jnp.float32),
                   jax.ShapeDtypeStruct((1, C), jnp.float32)),
    )(xk_g, xq)
    m0 = s0 / cnt
    v0 = ss0 / cnt - m0 * m0
    a0 = r2(gw0) * lax.rsqrt(v0 + 1e-5)
    c0 = r2(bw0) - m0 * a0

    y, s1, ss1 = pl.pallas_call(
        _y_body,
        grid=(nb,),
        in_specs=[kv_spec_k, xq_spec, c_spec(C), c_spec(C),
                  pl.BlockSpec((C, C // S), lambda i: (0, 0)),
                  c_spec(C // S)],
        out_specs=(pl.BlockSpec((NS * _QB, C // S), lambda i: (i, 0)),
                   c_spec(C // S), c_spec(C // S)),
        out_shape=(jax.ShapeDtypeStruct((N1 * NS, C // S), jnp.float32),
                   jax.ShapeDtypeStruct((1, C // S), jnp.float32),
                   jax.ShapeDtypeStruct((1, C // S), jnp.float32)),
    )(xk_g, xq, a0, c0, Ww1.T, r2(bw1))
    m1 = s1 / cnt
    v1 = ss1 / cnt - m1 * m1
    a1 = r2(gw1) * lax.rsqrt(v1 + 1e-5)
    c1 = r2(bew1) - m1 * a1

    return pl.pallas_call(
        _out_body,
        grid=(nb,),
        in_specs=[pl.BlockSpec((NS * _QB, C // S), lambda i: (i, 0)),
                  kv_spec_v, xq_spec, c_spec(C // S), c_spec(C // S),
                  pl.BlockSpec((C // S, C // S), lambda i: (0, 0)),
                  c_spec(C // S)],
        out_specs=pl.BlockSpec((_QB, C), lambda i: (i, 0)),
        out_shape=jax.ShapeDtypeStruct((N1, C), jnp.float32),
    )(y, xv_g, h1, a1, c1, Ww2.T, r2(bw2))


def kernel(p1, x1, p2, x2, knn_idx, interp_idx, W1, b1, g1, be1, W2, b2, g2,
           be2, Wq, bq, Wk, bk, Wv, bv, gw0, bw0, Ww1, bw1, gw1, bew1, Ww2,
           bw2):
    h1, xq, h2 = _dense_pre(x1, x2, W1, b1, g1, be1, W2, b2, g2, be2, Wq, bq)

    tbl = jnp.concatenate(
        [h2, jnp.pad(p2, ((0, 0), (0, 13)))], axis=1)
    gi = _sc_gather(tbl, _pad_idx(interp_idx.reshape(-1)))[:N1 * 3]
    xk, xv = _interp_kv(gi, p1, Wk, bk, Wv, bv)

    knn_flat = _pad_idx(knn_idx.reshape(-1))
    xk_g = _sc_gather(xk, knn_flat)[:N1 * NS]
    xv_g = _sc_gather(xv, knn_flat)[:N1 * NS]
    return _attention(xk_g, xv_g, xq, h1, gw0, bw0, Ww1, bw1, gw1, bew1,
                      Ww2, bw2)


# speedup vs baseline: 1.0122x; 1.0056x over previous
import functools
import jax
import jax.numpy as jnp
from jax import lax
from jax.experimental import pallas as pl
from jax.experimental.pallas import tpu as pltpu
from jax.experimental.pallas import tpu_sc as plsc

N1 = 50000
N2 = 12500
C = 64
NS = 16
S = 8

_BLK = 5000
_QB = 400
_CH = 128

_INFO = plsc.get_sparse_core_info()
_NW = _INFO.num_cores * _INFO.num_subcores


def _sc_gather(tables, idx):
    b = idx.shape[0]
    nt = len(tables)
    ds = [t.shape[1] for t in tables]
    per_w = b // _NW
    nch = per_w // _CH
    mesh = plsc.VectorSubcoreMesh(core_axis_name="c", subcore_axis_name="s")

    scratch = [pltpu.VMEM((_CH,), jnp.int32) for _ in range(2)]
    scratch += [pltpu.VMEM((_CH, d), jnp.float32) for d in ds for _ in range(2)]
    scratch += [pltpu.SemaphoreType.DMA] * 4

    @functools.partial(
        pl.kernel, mesh=mesh,
        out_type=tuple(jax.ShapeDtypeStruct((b, d), jnp.float32) for d in ds),
        compiler_params=pltpu.CompilerParams(use_tc_tiling_on_sc=False),
        scratch_types=scratch,
    )
    def k(*refs):
        tabs = refs[:nt]
        idx_hbm = refs[nt]
        outs = refs[nt + 1:2 * nt + 1]
        idxv = refs[2 * nt + 1:2 * nt + 3]
        rows = refs[2 * nt + 3:4 * nt + 3]
        gsem = refs[4 * nt + 3:4 * nt + 5]
        wsem = refs[4 * nt + 5:4 * nt + 7]
        wid = lax.axis_index("s") * _INFO.num_cores + lax.axis_index("c")
        base = wid * per_w

        def wb_copies(p, off):
            return [pltpu.make_async_copy(
                rows[2 * ti + p], outs[ti].at[pl.ds(off, _CH)], wsem[p])
                for ti in range(nt)]

        def g_copies(p):
            return [pltpu.make_async_copy(
                tabs[ti].at[idxv[p]], rows[2 * ti + p], gsem[p])
                for ti in range(nt)]

        def body(t, carry):
            offs = [base + (2 * t + p) * _CH for p in range(2)]
            for p in range(2):
                @pl.when(t > 0)
                def _(p=p, off=offs[p]):
                    for cp in wb_copies(p, off):
                        cp.wait()
                pltpu.sync_copy(idx_hbm.at[pl.ds(offs[p], _CH)], idxv[p])
                for cp in g_copies(p):
                    cp.start()
            for p in range(2):
                for cp in g_copies(p):
                    cp.wait()
                for cp in wb_copies(p, offs[p]):
                    cp.start()
            return carry

        lax.fori_loop(0, nch // 2, body, 0)
        last = [base + (nch - 2 + p) * _CH for p in range(2)]
        for p in range(2):
            for cp in wb_copies(p, last[p]):
                cp.wait()

    return k(*tables, idx)


def _pad_idx(idx_flat):
    b = idx_flat.shape[0]
    unit = 2 * _NW * _CH
    b_pad = ((b + unit - 1) // unit) * unit
    return jnp.pad(idx_flat, (0, b_pad - b))


def _gram_body(x_ref, gram_ref, csum_ref):
    i = pl.program_id(0)
    x = x_ref[...]
    g = lax.dot_general(x, x, (((0,), (0,)), ((), ())),
                        preferred_element_type=jnp.float32)
    s = jnp.sum(x, axis=0, keepdims=True)

    @pl.when(i == 0)
    def _():
        gram_ref[...] = jnp.zeros_like(gram_ref)
        csum_ref[...] = jnp.zeros_like(csum_ref)

    gram_ref[...] += g
    csum_ref[...] += s


def _lin_bn_stats(x, Wt, b):
    n, c = x.shape
    nb = n // _BLK
    gram, csum = pl.pallas_call(
        _gram_body,
        grid=(nb,),
        in_specs=[pl.BlockSpec((_BLK, c), lambda i: (i, 0))],
        out_specs=(pl.BlockSpec((c, c), lambda i: (0, 0)),
                   pl.BlockSpec((1, c), lambda i: (0, 0))),
        out_shape=(jax.ShapeDtypeStruct((c, c), jnp.float32),
                   jax.ShapeDtypeStruct((1, c), jnp.float32)),
    )(x)
    mean = (csum / n) @ Wt + b
    e2 = jnp.sum(Wt * (gram @ Wt), axis=0)[None] / n
    e2 = e2 + 2.0 * b * ((csum / n) @ Wt) + b * b
    var = e2 - mean * mean
    return mean, var


def _bn_fwd_body(x_ref, w1t_ref, a1_ref, c1_ref, wqt_ref, bq_ref,
                 h1_ref, xq_ref):
    h1 = jnp.maximum(
        jnp.dot(x_ref[...], w1t_ref[...], preferred_element_type=jnp.float32)
        * a1_ref[...] + c1_ref[...], 0.0)
    h1_ref[...] = h1
    xq_ref[...] = jnp.dot(h1, wqt_ref[...],
                          preferred_element_type=jnp.float32) + bq_ref[...]


def _h2_body(x2_ref, w2t_ref, b2_ref, g2_ref, be2_ref, h2_ref):
    z2 = jnp.dot(x2_ref[...], w2t_ref[...],
                 preferred_element_type=jnp.float32) + b2_ref[...]
    m2 = jnp.mean(z2, axis=0, keepdims=True)
    v2 = jnp.mean(z2 * z2, axis=0, keepdims=True) - m2 * m2
    h2_ref[...] = jnp.maximum(
        (z2 - m2) * (g2_ref[...] * lax.rsqrt(v2 + 1e-5)) + be2_ref[...], 0.0)


def _dense_pre(x1, x2, W1, b1, g1, be1, W2, b2, g2, be2, Wq, bq):
    n1, cin = x1.shape
    n2 = x2.shape[0]
    r2 = lambda a: a.reshape(1, -1)
    m1, v1 = _lin_bn_stats(x1, W1.T, r2(b1))
    a1 = r2(g1) * lax.rsqrt(v1 + 1e-5)
    c1 = r2(be1) - m1 * a1 + r2(b1) * a1
    nb = n1 // _BLK
    h1, xq = pl.pallas_call(
        _bn_fwd_body,
        grid=(nb,),
        in_specs=[
            pl.BlockSpec((_BLK, cin), lambda i: (i, 0)),
            pl.BlockSpec((cin, C), lambda i: (0, 0)),
            pl.BlockSpec((1, C), lambda i: (0, 0)),
            pl.BlockSpec((1, C), lambda i: (0, 0)),
            pl.BlockSpec((C, C), lambda i: (0, 0)),
            pl.BlockSpec((1, C), lambda i: (0, 0)),
        ],
        out_specs=(pl.BlockSpec((_BLK, C), lambda i: (i, 0)),
                   pl.BlockSpec((_BLK, C), lambda i: (i, 0))),
        out_shape=(jax.ShapeDtypeStruct((n1, C), jnp.float32),
                   jax.ShapeDtypeStruct((n1, C), jnp.float32)),
    )(x1, W1.T, a1, c1, Wq.T, r2(bq))
    h2 = pl.pallas_call(
        _h2_body,
        out_shape=jax.ShapeDtypeStruct((n2, C), jnp.float32),
    )(x2, W2.T, r2(b2), r2(g2), r2(be2))
    return h1, xq, h2


def _interp_body(g_ref, p1_ref, wkt_ref, bk_ref, wvt_ref, bv_ref,
                 xk_ref, xv_ref):
    g = g_ref[...].reshape(_QB, 3, 80)
    h2n = g[:, :, :C]
    p2n = g[:, :, C:C + 3]
    d = jnp.sqrt(jnp.sum((p1_ref[...][:, None, :] - p2n) ** 2, axis=-1))
    iw = 1.0 / (d + 1e-8)
    iw = iw / jnp.sum(iw, axis=-1, keepdims=True)
    x2i = jnp.sum(h2n * iw[:, :, None], axis=1)
    xk_ref[...] = jnp.dot(x2i, wkt_ref[...],
                          preferred_element_type=jnp.float32) + bk_ref[...]
    xv_ref[...] = jnp.dot(x2i, wvt_ref[...],
                          preferred_element_type=jnp.float32) + bv_ref[...]


def _interp_kv(g, p1, Wk, bk, Wv, bv):
    nb = N1 // _QB
    r2 = lambda a: a.reshape(1, -1)
    return pl.pallas_call(
        _interp_body,
        grid=(nb,),
        in_specs=[
            pl.BlockSpec((3 * _QB, 80), lambda i: (i, 0)),
            pl.BlockSpec((_QB, 3), lambda i: (i, 0)),
            pl.BlockSpec((C, C), lambda i: (0, 0)),
            pl.BlockSpec((1, C), lambda i: (0, 0)),
            pl.BlockSpec((C, C), lambda i: (0, 0)),
            pl.BlockSpec((1, C), lambda i: (0, 0)),
        ],
        out_specs=(pl.BlockSpec((_QB, C), lambda i: (i, 0)),
                   pl.BlockSpec((_QB, C), lambda i: (i, 0))),
        out_shape=(jax.ShapeDtypeStruct((N1, C), jnp.float32),
                   jax.ShapeDtypeStruct((N1, C), jnp.float32)),
    )(g, p1, Wk.T, r2(bk), Wv.T, r2(bv))


def _wstats_body(kv_ref, xq_ref, s_ref, ss_ref):
    i = pl.program_id(0)
    w = kv_ref[...].reshape(_QB, NS, C) - xq_ref[...][:, None, :]

    @pl.when(i == 0)
    def _():
        s_ref[...] = jnp.zeros_like(s_ref)
        ss_ref[...] = jnp.zeros_like(ss_ref)

    s_ref[...] += jnp.sum(w, axis=(0, 1))[None]
    ss_ref[...] += jnp.sum(w * w, axis=(0, 1))[None]


def _ystats_body(kv_ref, xq_ref, a0_ref, c0_ref, w1t_ref, bw1_ref,
                 s_ref, ss_ref):
    i = pl.program_id(0)
    w = kv_ref[...].reshape(_QB, NS, C) - xq_ref[...][:, None, :]
    u = jnp.maximum(w * a0_ref[...] + c0_ref[...], 0.0)
    y = jnp.dot(u.reshape(_QB * NS, C), w1t_ref[...],
                preferred_element_type=jnp.float32) + bw1_ref[...]

    @pl.when(i == 0)
    def _():
        s_ref[...] = jnp.zeros_like(s_ref)
        ss_ref[...] = jnp.zeros_like(ss_ref)

    s_ref[...] += jnp.sum(y, axis=0, keepdims=True)
    ss_ref[...] += jnp.sum(y * y, axis=0, keepdims=True)


def _out_body(xk_ref, kv_ref, xq_ref, h1_ref, a0_ref, c0_ref, w1t_ref,
              bw1_ref, a1_ref, c1_ref, w2t_ref, bw2_ref, o_ref):
    w = xk_ref[...].reshape(_QB, NS, C) - xq_ref[...][:, None, :]
    u = jnp.maximum(w * a0_ref[...] + c0_ref[...], 0.0)
    y = jnp.dot(u.reshape(_QB * NS, C), w1t_ref[...],
                preferred_element_type=jnp.float32) + bw1_ref[...]
    v = jnp.maximum(y * a1_ref[...] + c1_ref[...], 0.0)
    z = jnp.dot(v, w2t_ref[...],
                preferred_element_type=jnp.float32) + bw2_ref[...]
    z = z.reshape(_QB, NS, C // S)
    z = z - jnp.max(z, axis=1, keepdims=True)
    e = jnp.exp(z)
    wsm = e / jnp.sum(e, axis=1, keepdims=True)
    xv = kv_ref[...].reshape(_QB, NS, S, C // S)
    x2o = jnp.sum(xv * wsm[:, :, None, :], axis=1).reshape(_QB, C)
    o_ref[...] = h1_ref[...] + x2o


def _attention(xk_g, xv_g, xq, h1, gw0, bw0, Ww1, bw1, gw1, bew1, Ww2, bw2):
    nb = N1 // _QB
    r2 = lambda a: a.reshape(1, -1)
    cnt = float(N1 * NS)

    kv_spec_k = pl.BlockSpec((NS * _QB, C), lambda i: (i, 0))
    kv_spec_v = pl.BlockSpec((NS * _QB, C), lambda i: (i, 0))
    xq_spec = pl.BlockSpec((_QB, C), lambda i: (i, 0))
    c_spec = lambda c: pl.BlockSpec((1, c), lambda i: (0, 0))

    s0, ss0 = pl.pallas_call(
        _wstats_body,
        grid=(nb,),
        in_specs=[kv_spec_k, xq_spec],
        out_specs=(c_spec(C), c_spec(C)),
        out_shape=(jax.ShapeDtypeStruct((1, C), jnp.float32),
                   jax.ShapeDtypeStruct((1, C), jnp.float32)),
    )(xk_g, xq)
    m0 = s0 / cnt
    v0 = ss0 / cnt - m0 * m0
    a0 = r2(gw0) * lax.rsqrt(v0 + 1e-5)
    c0 = r2(bw0) - m0 * a0

    s1, ss1 = pl.pallas_call(
        _ystats_body,
        grid=(nb,),
        in_specs=[kv_spec_k, xq_spec, c_spec(C), c_spec(C),
                  pl.BlockSpec((C, C // S), lambda i: (0, 0)),
                  c_spec(C // S)],
        out_specs=(c_spec(C // S), c_spec(C // S)),
        out_shape=(jax.ShapeDtypeStruct((1, C // S), jnp.float32),
                   jax.ShapeDtypeStruct((1, C // S), jnp.float32)),
    )(xk_g, xq, a0, c0, Ww1.T, r2(bw1))
    m1 = s1 / cnt
    v1 = ss1 / cnt - m1 * m1
    a1 = r2(gw1) * lax.rsqrt(v1 + 1e-5)
    c1 = r2(bew1) - m1 * a1

    return pl.pallas_call(
        _out_body,
        grid=(nb,),
        in_specs=[kv_spec_k, kv_spec_v, xq_spec, xq_spec,
                  c_spec(C), c_spec(C),
                  pl.BlockSpec((C, C // S), lambda i: (0, 0)),
                  c_spec(C // S), c_spec(C // S), c_spec(C // S),
                  pl.BlockSpec((C // S, C // S), lambda i: (0, 0)),
                  c_spec(C // S)],
        out_specs=pl.BlockSpec((_QB, C), lambda i: (i, 0)),
        out_shape=jax.ShapeDtypeStruct((N1, C), jnp.float32),
    )(xk_g, xv_g, xq, h1, a0, c0, Ww1.T, r2(bw1), a1, c1, Ww2.T, r2(bw2))


def kernel(p1, x1, p2, x2, knn_idx, interp_idx, W1, b1, g1, be1, W2, b2, g2,
           be2, Wq, bq, Wk, bk, Wv, bv, gw0, bw0, Ww1, bw1, gw1, bew1, Ww2,
           bw2):
    h1, xq, h2 = _dense_pre(x1, x2, W1, b1, g1, be1, W2, b2, g2, be2, Wq, bq)

    tbl = jnp.concatenate(
        [h2, jnp.pad(p2, ((0, 0), (0, 13)))], axis=1)
    (gi,) = _sc_gather([tbl], _pad_idx(interp_idx.reshape(-1)))
    xk, xv = _interp_kv(gi[:N1 * 3], p1, Wk, bk, Wv, bv)

    xk_g, xv_g = _sc_gather([xk, xv], _pad_idx(knn_idx.reshape(-1)))
    xk_g = xk_g[:N1 * NS]
    xv_g = xv_g[:N1 * NS]
    return _attention(xk_g, xv_g, xq, h1, gw0, bw0, Ww1, bw1, gw1, bew1,
                      Ww2, bw2)


# speedup vs baseline: 1.0122x; 1.0000x over previous
import functools
import jax
import jax.numpy as jnp
from jax import lax
from jax.experimental import pallas as pl
from jax.experimental.pallas import tpu as pltpu
from jax.experimental.pallas import tpu_sc as plsc

N1 = 50000
N2 = 12500
C = 64
NS = 16
S = 8

_BLK = 5000
_QB = 400
_CH = 128

_INFO = plsc.get_sparse_core_info()
_NW = _INFO.num_cores * _INFO.num_subcores


def _sc_gather(tables, idx):
    b = idx.shape[0]
    nt = len(tables)
    ds = [t.shape[1] for t in tables]
    per_w = b // _NW
    nch = per_w // _CH
    mesh = plsc.VectorSubcoreMesh(core_axis_name="c", subcore_axis_name="s")

    scratch = [pltpu.VMEM((_CH,), jnp.int32) for _ in range(2)]
    scratch += [pltpu.VMEM((_CH, d), jnp.float32) for d in ds for _ in range(2)]
    scratch += [pltpu.SemaphoreType.DMA] * 4

    @functools.partial(
        pl.kernel, mesh=mesh,
        out_type=tuple(jax.ShapeDtypeStruct((b, d), jnp.float32) for d in ds),
        compiler_params=pltpu.CompilerParams(use_tc_tiling_on_sc=False),
        scratch_types=scratch,
    )
    def k(*refs):
        tabs = refs[:nt]
        idx_hbm = refs[nt]
        outs = refs[nt + 1:2 * nt + 1]
        idxv = refs[2 * nt + 1:2 * nt + 3]
        rows = refs[2 * nt + 3:4 * nt + 3]
        gsem = refs[4 * nt + 3:4 * nt + 5]
        wsem = refs[4 * nt + 5:4 * nt + 7]
        wid = lax.axis_index("s") * _INFO.num_cores + lax.axis_index("c")
        base = wid * per_w

        def wb_copies(p, off):
            return [pltpu.make_async_copy(
                rows[2 * ti + p], outs[ti].at[pl.ds(off, _CH)], wsem[p])
                for ti in range(nt)]

        def g_copies(p):
            return [pltpu.make_async_copy(
                tabs[ti].at[idxv[p]], rows[2 * ti + p], gsem[p])
                for ti in range(nt)]

        def body(t, carry):
            offs = [base + (2 * t + p) * _CH for p in range(2)]
            for p in range(2):
                @pl.when(t > 0)
                def _(p=p, off=offs[p]):
                    for cp in wb_copies(p, off):
                        cp.wait()
                pltpu.sync_copy(idx_hbm.at[pl.ds(offs[p], _CH)], idxv[p])
                for cp in g_copies(p):
                    cp.start()
            for p in range(2):
                for cp in g_copies(p):
                    cp.wait()
                for cp in wb_copies(p, offs[p]):
                    cp.start()
            return carry

        lax.fori_loop(0, nch // 2, body, 0)
        last = [base + (nch - 2 + p) * _CH for p in range(2)]
        for p in range(2):
            for cp in wb_copies(p, last[p]):
                cp.wait()

    return k(*tables, idx)


def _pad_idx(idx_flat):
    b = idx_flat.shape[0]
    unit = 2 * _NW * _CH
    b_pad = ((b + unit - 1) // unit) * unit
    return jnp.pad(idx_flat, (0, b_pad - b))


def _gram_body(x_ref, gram_ref, csum_ref):
    i = pl.program_id(0)
    x = x_ref[...]
    g = lax.dot_general(x, x, (((0,), (0,)), ((), ())),
                        preferred_element_type=jnp.float32)
    s = jnp.sum(x, axis=0, keepdims=True)

    @pl.when(i == 0)
    def _():
        gram_ref[...] = jnp.zeros_like(gram_ref)
        csum_ref[...] = jnp.zeros_like(csum_ref)

    gram_ref[...] += g
    csum_ref[...] += s


def _lin_bn_stats(x, Wt, b):
    n, c = x.shape
    nb = n // _BLK
    gram, csum = pl.pallas_call(
        _gram_body,
        grid=(nb,),
        in_specs=[pl.BlockSpec((_BLK, c), lambda i: (i, 0))],
        out_specs=(pl.BlockSpec((c, c), lambda i: (0, 0)),
                   pl.BlockSpec((1, c), lambda i: (0, 0))),
        out_shape=(jax.ShapeDtypeStruct((c, c), jnp.float32),
                   jax.ShapeDtypeStruct((1, c), jnp.float32)),
    )(x)
    mean = (csum / n) @ Wt + b
    e2 = jnp.sum(Wt * (gram @ Wt), axis=0)[None] / n
    e2 = e2 + 2.0 * b * ((csum / n) @ Wt) + b * b
    var = e2 - mean * mean
    return mean, var


def _bn_fwd_body(x_ref, w1t_ref, a1_ref, c1_ref, wqt_ref, bq_ref,
                 h1_ref, xq_ref):
    h1 = jnp.maximum(
        jnp.dot(x_ref[...], w1t_ref[...], preferred_element_type=jnp.float32)
        * a1_ref[...] + c1_ref[...], 0.0)
    h1_ref[...] = h1
    xq_ref[...] = jnp.dot(h1, wqt_ref[...],
                          preferred_element_type=jnp.float32) + bq_ref[...]


def _h2_body(x2_ref, w2t_ref, b2_ref, g2_ref, be2_ref, h2_ref):
    z2 = jnp.dot(x2_ref[...], w2t_ref[...],
                 preferred_element_type=jnp.float32) + b2_ref[...]
    m2 = jnp.mean(z2, axis=0, keepdims=True)
    v2 = jnp.mean(z2 * z2, axis=0, keepdims=True) - m2 * m2
    h2_ref[...] = jnp.maximum(
        (z2 - m2) * (g2_ref[...] * lax.rsqrt(v2 + 1e-5)) + be2_ref[...], 0.0)


def _h1_xq(x1, W1, b1, g1, be1, Wq, bq):
    n1, cin = x1.shape
    r2 = lambda a: a.reshape(1, -1)
    m1, v1 = _lin_bn_stats(x1, W1.T, r2(b1))
    a1 = r2(g1) * lax.rsqrt(v1 + 1e-5)
    c1 = r2(be1) - m1 * a1 + r2(b1) * a1
    nb = n1 // _BLK
    h1, xq = pl.pallas_call(
        _bn_fwd_body,
        grid=(nb,),
        in_specs=[
            pl.BlockSpec((_BLK, cin), lambda i: (i, 0)),
            pl.BlockSpec((cin, C), lambda i: (0, 0)),
            pl.BlockSpec((1, C), lambda i: (0, 0)),
            pl.BlockSpec((1, C), lambda i: (0, 0)),
            pl.BlockSpec((C, C), lambda i: (0, 0)),
            pl.BlockSpec((1, C), lambda i: (0, 0)),
        ],
        out_specs=(pl.BlockSpec((_BLK, C), lambda i: (i, 0)),
                   pl.BlockSpec((_BLK, C), lambda i: (i, 0))),
        out_shape=(jax.ShapeDtypeStruct((n1, C), jnp.float32),
                   jax.ShapeDtypeStruct((n1, C), jnp.float32)),
    )(x1, W1.T, a1, c1, Wq.T, r2(bq))
    return h1, xq


def _h2(x2, W2, b2, g2, be2):
    r2 = lambda a: a.reshape(1, -1)
    return pl.pallas_call(
        _h2_body,
        out_shape=jax.ShapeDtypeStruct((x2.shape[0], C), jnp.float32),
    )(x2, W2.T, r2(b2), r2(g2), r2(be2))


def _interp_body(g_ref, p1_ref, wkt_ref, bk_ref, wvt_ref, bv_ref,
                 xk_ref, xv_ref):
    g = g_ref[...].reshape(_QB, 3, 80)
    h2n = g[:, :, :C]
    p2n = g[:, :, C:C + 3]
    d = jnp.sqrt(jnp.sum((p1_ref[...][:, None, :] - p2n) ** 2, axis=-1))
    iw = 1.0 / (d + 1e-8)
    iw = iw / jnp.sum(iw, axis=-1, keepdims=True)
    x2i = jnp.sum(h2n * iw[:, :, None], axis=1)
    xk_ref[...] = jnp.dot(x2i, wkt_ref[...],
                          preferred_element_type=jnp.float32) + bk_ref[...]
    xv_ref[...] = jnp.dot(x2i, wvt_ref[...],
                          preferred_element_type=jnp.float32) + bv_ref[...]


def _interp_kv(g, p1, Wk, bk, Wv, bv):
    nb = N1 // _QB
    r2 = lambda a: a.reshape(1, -1)
    return pl.pallas_call(
        _interp_body,
        grid=(nb,),
        in_specs=[
            pl.BlockSpec((3 * _QB, 80), lambda i: (i, 0)),
            pl.BlockSpec((_QB, 3), lambda i: (i, 0)),
            pl.BlockSpec((C, C), lambda i: (0, 0)),
            pl.BlockSpec((1, C), lambda i: (0, 0)),
            pl.BlockSpec((C, C), lambda i: (0, 0)),
            pl.BlockSpec((1, C), lambda i: (0, 0)),
        ],
        out_specs=(pl.BlockSpec((_QB, C), lambda i: (i, 0)),
                   pl.BlockSpec((_QB, C), lambda i: (i, 0))),
        out_shape=(jax.ShapeDtypeStruct((N1, C), jnp.float32),
                   jax.ShapeDtypeStruct((N1, C), jnp.float32)),
    )(g, p1, Wk.T, r2(bk), Wv.T, r2(bv))


def _wstats_body(kv_ref, xq_ref, s_ref, ss_ref):
    i = pl.program_id(0)
    w = kv_ref[...].reshape(_QB, NS, C) - xq_ref[...][:, None, :]

    @pl.when(i == 0)
    def _():
        s_ref[...] = jnp.zeros_like(s_ref)
        ss_ref[...] = jnp.zeros_like(ss_ref)

    s_ref[...] += jnp.sum(w, axis=(0, 1))[None]
    ss_ref[...] += jnp.sum(w * w, axis=(0, 1))[None]


def _ystats_body(kv_ref, xq_ref, a0_ref, c0_ref, w1t_ref, bw1_ref,
                 s_ref, ss_ref):
    i = pl.program_id(0)
    w = kv_ref[...].reshape(_QB, NS, C) - xq_ref[...][:, None, :]
    u = jnp.maximum(w * a0_ref[...] + c0_ref[...], 0.0)
    y = jnp.dot(u.reshape(_QB * NS, C), w1t_ref[...],
                preferred_element_type=jnp.float32) + bw1_ref[...]

    @pl.when(i == 0)
    def _():
        s_ref[...] = jnp.zeros_like(s_ref)
        ss_ref[...] = jnp.zeros_like(ss_ref)

    s_ref[...] += jnp.sum(y, axis=0, keepdims=True)
    ss_ref[...] += jnp.sum(y * y, axis=0, keepdims=True)


def _out_body(xk_ref, kv_ref, xq_ref, h1_ref, a0_ref, c0_ref, w1t_ref,
              bw1_ref, a1_ref, c1_ref, w2t_ref, bw2_ref, o_ref):
    w = xk_ref[...].reshape(_QB, NS, C) - xq_ref[...][:, None, :]
    u = jnp.maximum(w * a0_ref[...] + c0_ref[...], 0.0)
    y = jnp.dot(u.reshape(_QB * NS, C), w1t_ref[...],
                preferred_element_type=jnp.float32) + bw1_ref[...]
    v = jnp.maximum(y * a1_ref[...] + c1_ref[...], 0.0)
    z = jnp.dot(v, w2t_ref[...],
                preferred_element_type=jnp.float32) + bw2_ref[...]
    z = z.reshape(_QB, NS, C // S)
    z = z - jnp.max(z, axis=1, keepdims=True)
    e = jnp.exp(z)
    wsm = e / jnp.sum(e, axis=1, keepdims=True)
    xv = kv_ref[...].reshape(_QB, NS, S, C // S)
    x2o = jnp.sum(xv * wsm[:, :, None, :], axis=1).reshape(_QB, C)
    o_ref[...] = h1_ref[...] + x2o


def _attention(xk_g, xv_g, xq, h1, gw0, bw0, Ww1, bw1, gw1, bew1, Ww2, bw2):
    nb = N1 // _QB
    r2 = lambda a: a.reshape(1, -1)
    cnt = float(N1 * NS)

    kv_spec_k = pl.BlockSpec((NS * _QB, C), lambda i: (i, 0))
    kv_spec_v = pl.BlockSpec((NS * _QB, C), lambda i: (i, 0))
    xq_spec = pl.BlockSpec((_QB, C), lambda i: (i, 0))
    c_spec = lambda c: pl.BlockSpec((1, c), lambda i: (0, 0))

    s0, ss0 = pl.pallas_call(
        _wstats_body,
        grid=(nb,),
        in_specs=[kv_spec_k, xq_spec],
        out_specs=(c_spec(C), c_spec(C)),
        out_shape=(jax.ShapeDtypeStruct((1, C), jnp.float32),
                   jax.ShapeDtypeStruct((1, C), jnp.float32)),
    )(xk_g, xq)
    m0 = s0 / cnt
    v0 = ss0 / cnt - m0 * m0
    a0 = r2(gw0) * lax.rsqrt(v0 + 1e-5)
    c0 = r2(bw0) - m0 * a0

    s1, ss1 = pl.pallas_call(
        _ystats_body,
        grid=(nb,),
        in_specs=[kv_spec_k, xq_spec, c_spec(C), c_spec(C),
                  pl.BlockSpec((C, C // S), lambda i: (0, 0)),
                  c_spec(C // S)],
        out_specs=(c_spec(C // S), c_spec(C // S)),
        out_shape=(jax.ShapeDtypeStruct((1, C // S), jnp.float32),
                   jax.ShapeDtypeStruct((1, C // S), jnp.float32)),
    )(xk_g, xq, a0, c0, Ww1.T, r2(bw1))
    m1 = s1 / cnt
    v1 = ss1 / cnt - m1 * m1
    a1 = r2(gw1) * lax.rsqrt(v1 + 1e-5)
    c1 = r2(bew1) - m1 * a1

    return pl.pallas_call(
        _out_body,
        grid=(nb,),
        in_specs=[kv_spec_k, kv_spec_v, xq_spec, xq_spec,
                  c_spec(C), c_spec(C),
                  pl.BlockSpec((C, C // S), lambda i: (0, 0)),
                  c_spec(C // S), c_spec(C // S), c_spec(C // S),
                  pl.BlockSpec((C // S, C // S), lambda i: (0, 0)),
                  c_spec(C // S)],
        out_specs=pl.BlockSpec((_QB, C), lambda i: (i, 0)),
        out_shape=jax.ShapeDtypeStruct((N1, C), jnp.float32),
    )(xk_g, xv_g, xq, h1, a0, c0, Ww1.T, r2(bw1), a1, c1, Ww2.T, r2(bw2))


def kernel(p1, x1, p2, x2, knn_idx, interp_idx, W1, b1, g1, be1, W2, b2, g2,
           be2, Wq, bq, Wk, bk, Wv, bv, gw0, bw0, Ww1, bw1, gw1, bew1, Ww2,
           bw2):
    h2 = _h2(x2, W2, b2, g2, be2)
    tbl = jnp.concatenate(
        [h2, jnp.pad(p2, ((0, 0), (0, 13)))], axis=1)
    (gi,) = _sc_gather([tbl], _pad_idx(interp_idx.reshape(-1)))
    h1, xq = _h1_xq(x1, W1, b1, g1, be1, Wq, bq)
    xk, xv = _interp_kv(gi[:N1 * 3], p1, Wk, bk, Wv, bv)

    xk_g, xv_g = _sc_gather([xk, xv], _pad_idx(knn_idx.reshape(-1)))
    xk_g = xk_g[:N1 * NS]
    xv_g = xv_g[:N1 * NS]
    return _attention(xk_g, xv_g, xq, h1, gw0, bw0, Ww1, bw1, gw1, bew1,
                      Ww2, bw2)
